# baseline (speedup 1.0000x reference)
import jax
import jax.numpy as jnp
from jax import lax
from jax.experimental import pallas as pl
from jax.experimental.pallas import tpu as pltpu
from jax.experimental.pallas import tpu_sc as plsc

N = 50000
D = 256
G = 1024
GH = 2 * D

ABLK = 400
NAB = 125

SEG_PER_TILE = G // 32
CHUNK = 80
NGRP = CHUNK // 16


def _stage_a_body(seg_ref, hv_ref, wg_ref, bg_ref, w_ref, g_ref, bnd_ref,
                  cnt_ref):
    i = pl.program_id(0)

    @pl.when(i == 0)
    def _init():
        cnt_ref[...] = jnp.zeros_like(cnt_ref)

    hv = hv_ref[...]
    wg = wg_ref[...]
    logits = jnp.sum(hv * wg, axis=1, keepdims=True) + bg_ref[0, 0]
    gate = 1.0 / (1.0 + jnp.exp(-logits))
    w_ref[...] = gate * hv
    g_ref[...] = jnp.broadcast_to(gate, (ABLK, 16))

    seg = seg_ref[0]
    thr = lax.broadcasted_iota(jnp.int32, (1, 128), 1) * SEG_PER_TILE
    below = (seg < thr).astype(jnp.float32)
    cnt_ref[...] += jnp.sum(below, axis=0, keepdims=True)

    @pl.when(i == NAB - 1)
    def _fin():
        bnd_ref[...] = cnt_ref[...]


def _bcast_lane(vec, lane):
    idx = jnp.full((16, 1), lane, jnp.int32)
    return lax.gather(
        vec, idx,
        dimension_numbers=lax.GatherDimensionNumbers(
            offset_dims=(), collapsed_slice_dims=(0,), start_index_map=(0,)),
        slice_sizes=(1,),
        mode=lax.GatherScatterMode.PROMISE_IN_BOUNDS)


def _sc_body(w_hbm, g_hbm, ids_hbm, bnd_hbm, pw_hbm, pc_hbm,
             wbuf, gbuf, idsv, bndv, acc, cacc):
    cid = lax.axis_index("c")
    sid = lax.axis_index("s")
    t = sid * 2 + cid
    iota16 = lax.iota(jnp.int32, 16)

    def zs(s, c0):
        for k in range(16):
            acc[s, pl.ds(k * 16, 16)] = jnp.zeros((16,), jnp.float32)
        cacc[s, :] = jnp.zeros((16,), jnp.float32)
        return c0

    lax.fori_loop(0, SEG_PER_TILE, zs, 0)

    pltpu.sync_copy(bnd_hbm, bndv)

    def extract(idx):
        tot = jnp.float32(0.0)
        for j in range(3):
            v = bndv[0, pl.ds(j * 16, 16)]
            tot += jnp.sum(jnp.where(iota16 + j * 16 == idx, v, 0.0))
        return tot.astype(jnp.int32)

    lo = extract(t)
    hi = extract(t + 1)
    seg0 = t * SEG_PER_TILE

    zero = jnp.zeros((16,), jnp.float32)

    def chunk(q, regs):
        off = q * CHUNK
        pltpu.sync_copy(ids_hbm.at[pl.ds(off, CHUNK)], idsv.at[pl.ds(0, CHUNK)])
        idsv[pl.ds(CHUNK, 16)] = jnp.full((16,), G, jnp.int32)
        pltpu.sync_copy(w_hbm.at[pl.ds(off, CHUNK)], wbuf)
        pltpu.sync_copy(g_hbm.at[pl.ds(off, CHUNK)], gbuf)

        def group(m, regs):
            base = m * 16
            vc = idsv[pl.ds(base, 16)]
            vn = idsv[pl.ds(base + 16, 16)]
            bcs = [_bcast_lane(vc, l) for l in range(16)]
            bcs.append(_bcast_lane(vn, 0))
            for l in range(16):
                jj = base + l
                cur = bcs[l]
                changed = cur != bcs[l + 1]
                loc = cur - seg0
                mfl = changed & (loc >= 0) & (loc < SEG_PER_TILE)
                nregs = []
                for k in range(16):
                    r = regs[k] + wbuf[jj, pl.ds(k * 16, 16)]
                    plsc.addupdate_scatter(
                        acc, [loc, iota16 + k * 16], r, mask=mfl)
                    nregs.append(jnp.where(changed, zero, r))
                rg = regs[16] + gbuf[jj]
                plsc.addupdate_scatter(cacc, [loc, iota16], rg, mask=mfl)
                nregs.append(jnp.where(changed, zero, rg))
                regs = tuple(nregs)
            return regs

        return lax.fori_loop(0, NGRP, group, regs)

    lax.fori_loop(lo // CHUNK, (hi + CHUNK - 1) // CHUNK, chunk, (zero,) * 17)

    pltpu.sync_copy(acc, pw_hbm.at[pl.ds(t * SEG_PER_TILE, SEG_PER_TILE)])
    pltpu.sync_copy(cacc, pc_hbm.at[pl.ds(t * SEG_PER_TILE, SEG_PER_TILE)])


def _final_body(pw_ref, pc_ref, wp_ref, bp_ref, out_ref):
    s = pw_ref[...]
    c = pc_ref[:, 0:1]
    out_ref[...] = lax.dot_general(
        s, wp_ref[...], (((1,), (1,)), ((), ())),
        preferred_element_type=jnp.float32) + c * bp_ref[...]


def kernel(hv, segment_ids, W_gate, b_gate, W_proj, b_proj):
    bg = b_gate.reshape(1, 1)
    bp = b_proj.reshape(1, GH)
    ids = segment_ids.astype(jnp.int32)
    seg3 = ids.reshape(NAB, ABLK, 1)

    w, g16, bnd = pl.pallas_call(
        _stage_a_body,
        grid=(NAB,),
        in_specs=[
            pl.BlockSpec((1, ABLK, 1), lambda i: (i, 0, 0)),
            pl.BlockSpec((ABLK, D), lambda i: (i, 0)),
            pl.BlockSpec((1, D), lambda i: (0, 0)),
            pl.BlockSpec((1, 1), lambda i: (0, 0)),
        ],
        out_specs=[
            pl.BlockSpec((ABLK, D), lambda i: (i, 0)),
            pl.BlockSpec((ABLK, 16), lambda i: (i, 0)),
            pl.BlockSpec((1, 128), lambda i: (0, 0)),
        ],
        out_shape=[
            jax.ShapeDtypeStruct((N, D), jnp.float32),
            jax.ShapeDtypeStruct((N, 16), jnp.float32),
            jax.ShapeDtypeStruct((1, 128), jnp.float32),
        ],
        scratch_shapes=[pltpu.VMEM((1, 128), jnp.float32)],
    )(seg3, hv, W_gate, bg)

    mesh = plsc.VectorSubcoreMesh(core_axis_name="c", subcore_axis_name="s")
    pw, pc = pl.kernel(
        _sc_body,
        out_type=[
            jax.ShapeDtypeStruct((G, D), jnp.float32),
            jax.ShapeDtypeStruct((G, 16), jnp.float32),
        ],
        mesh=mesh,
        compiler_params=pltpu.CompilerParams(needs_layout_passes=False),
        scratch_types=[
            pltpu.VMEM((CHUNK, D), jnp.float32),
            pltpu.VMEM((CHUNK, 16), jnp.float32),
            pltpu.VMEM((CHUNK + 16,), jnp.int32),
            pltpu.VMEM((1, 128), jnp.float32),
            pltpu.VMEM((SEG_PER_TILE, D), jnp.float32),
            pltpu.VMEM((SEG_PER_TILE, 16), jnp.float32),
        ],
    )(w, g16, ids, bnd)

    out = pl.pallas_call(
        _final_body,
        grid=(1,),
        in_specs=[
            pl.BlockSpec((G, D), lambda i: (0, 0)),
            pl.BlockSpec((G, 16), lambda i: (0, 0)),
            pl.BlockSpec((GH, D), lambda i: (0, 0)),
            pl.BlockSpec((1, GH), lambda i: (0, 0)),
        ],
        out_specs=pl.BlockSpec((G, GH), lambda i: (0, 0)),
        out_shape=jax.ShapeDtypeStruct((G, GH), jnp.float32),
    )(pw, pc, W_proj, bp)
    return out


# speedup vs baseline: 1.8754x; 1.8754x over previous
import jax
import jax.numpy as jnp
from jax import lax
from jax.experimental import pallas as pl
from jax.experimental.pallas import tpu as pltpu
from jax.experimental.pallas import tpu_sc as plsc

N = 50000
D = 256
G = 1024
GH = 2 * D

SEG_PER_TILE = G // 32
CHUNK = 80
NGRP = CHUNK // 16
NIDS = 50176
NSENT = (NIDS - N) // 16


def _bcast_lane(vec, lane):
    idx = jnp.full((16, 1), lane, jnp.int32)
    return lax.gather(
        vec, idx,
        dimension_numbers=lax.GatherDimensionNumbers(
            offset_dims=(), collapsed_slice_dims=(0,), start_index_map=(0,)),
        slice_sizes=(1,),
        mode=lax.GatherScatterMode.PROMISE_IN_BOUNDS)


def _sc_body(hv_hbm, ids_hbm, wg_hbm, bg_hbm, pw_hbm, pc_hbm,
             hvbuf, idsf, wgv, bgv, acc, cacc):
    cid = lax.axis_index("c")
    sid = lax.axis_index("s")
    t = sid * 2 + cid
    iota16 = lax.iota(jnp.int32, 16)
    fzero = jnp.zeros((16,), jnp.float32)

    def zs(s, c0):
        for k in range(16):
            acc[s, pl.ds(k * 16, 16)] = fzero
        cacc[s, :] = fzero
        return c0

    lax.fori_loop(0, SEG_PER_TILE, zs, 0)

    pltpu.sync_copy(ids_hbm, idsf.at[pl.ds(0, N)])

    def sent(i, c0):
        idsf[pl.ds(N + i * 16, 16)] = jnp.full((16,), G, jnp.int32)
        return c0

    lax.fori_loop(0, NSENT, sent, 0)
    pltpu.sync_copy(wg_hbm, wgv)
    pltpu.sync_copy(bg_hbm, bgv)

    thr_lo = t * SEG_PER_TILE
    thr_hi = thr_lo + SEG_PER_TILE

    def cnt(i, carry):
        v = idsf[pl.ds(i * 16, 16)]
        cl, ch = carry
        cl = cl + jnp.where(v < thr_lo, 1.0, 0.0)
        ch = ch + jnp.where(v < thr_hi, 1.0, 0.0)
        return (cl, ch)

    clv, chv = lax.fori_loop(0, NIDS // 16, cnt, (fzero, fzero))
    lo = jnp.sum(clv).astype(jnp.int32)
    hi = jnp.sum(chv).astype(jnp.int32)
    seg0 = thr_lo

    wgs = [wgv[pl.ds(k * 16, 16)] for k in range(16)]
    bg0 = bgv[...]

    def chunk(q, regs):
        off = q * CHUNK
        pltpu.sync_copy(hv_hbm.at[pl.ds(off, CHUNK)], hvbuf)

        def group(m, regs):
            base = off + m * 16
            vc = idsf[pl.ds(base, 16)]
            vn = idsf[pl.ds(base + 16, 16)]
            bcs = [_bcast_lane(vc, l) for l in range(16)]
            bcs.append(_bcast_lane(vn, 0))
            for l in range(16):
                jj = m * 16 + l
                rows = [hvbuf[jj, pl.ds(k * 16, 16)] for k in range(16)]
                prods = [rows[k] * wgs[k] for k in range(16)]
                for step in (8, 4, 2, 1):
                    prods = [prods[i2] + prods[i2 + step]
                             for i2 in range(step)]
                x = jnp.broadcast_to(jnp.sum(prods[0]), (16,)) + bg0
                gate = 1.0 / (1.0 + jnp.exp(-x))

                cur = bcs[l]
                changed = cur != bcs[l + 1]
                loc = cur - seg0
                mfl = changed & (loc >= 0) & (loc < SEG_PER_TILE)
                nregs = []
                for k in range(16):
                    r = regs[k] + rows[k] * gate
                    plsc.addupdate_scatter(
                        acc, [loc, iota16 + k * 16], r, mask=mfl)
                    nregs.append(jnp.where(changed, fzero, r))
                rg = regs[16] + gate
                plsc.addupdate_scatter(cacc, [loc, iota16], rg, mask=mfl)
                nregs.append(jnp.where(changed, fzero, rg))
                regs = tuple(nregs)
            return regs

        return lax.fori_loop(0, NGRP, group, regs)

    lax.fori_loop(lo // CHUNK, (hi + CHUNK - 1) // CHUNK, chunk, (fzero,) * 17)

    pltpu.sync_copy(acc, pw_hbm.at[pl.ds(t * SEG_PER_TILE, SEG_PER_TILE)])
    pltpu.sync_copy(cacc, pc_hbm.at[pl.ds(t * SEG_PER_TILE, SEG_PER_TILE)])


def _final_body(pw_ref, pc_ref, wp_ref, bp_ref, out_ref):
    s = pw_ref[...]
    c = pc_ref[:, 0:1]
    out_ref[...] = lax.dot_general(
        s, wp_ref[...], (((1,), (1,)), ((), ())),
        preferred_element_type=jnp.float32) + c * bp_ref[...]


def kernel(hv, segment_ids, W_gate, b_gate, W_proj, b_proj):
    bp = b_proj.reshape(1, GH)
    ids = segment_ids.astype(jnp.int32)
    wg = W_gate.reshape(D)
    bg16 = jnp.broadcast_to(b_gate, (16,))

    mesh = plsc.VectorSubcoreMesh(core_axis_name="c", subcore_axis_name="s")
    pw, pc = pl.kernel(
        _sc_body,
        out_type=[
            jax.ShapeDtypeStruct((G, D), jnp.float32),
            jax.ShapeDtypeStruct((G, 16), jnp.float32),
        ],
        mesh=mesh,
        compiler_params=pltpu.CompilerParams(needs_layout_passes=False),
        scratch_types=[
            pltpu.VMEM((CHUNK, D), jnp.float32),
            pltpu.VMEM((NIDS,), jnp.int32),
            pltpu.VMEM((D,), jnp.float32),
            pltpu.VMEM((16,), jnp.float32),
            pltpu.VMEM((SEG_PER_TILE, D), jnp.float32),
            pltpu.VMEM((SEG_PER_TILE, 16), jnp.float32),
        ],
    )(hv, ids, wg, bg16)

    out = pl.pallas_call(
        _final_body,
        grid=(1,),
        in_specs=[
            pl.BlockSpec((G, D), lambda i: (0, 0)),
            pl.BlockSpec((G, 16), lambda i: (0, 0)),
            pl.BlockSpec((GH, D), lambda i: (0, 0)),
            pl.BlockSpec((1, GH), lambda i: (0, 0)),
        ],
        out_specs=pl.BlockSpec((G, GH), lambda i: (0, 0)),
        out_shape=jax.ShapeDtypeStruct((G, GH), jnp.float32),
    )(pw, pc, W_proj, bp)
    return out


# speedup vs baseline: 2.2712x; 1.2111x over previous
import jax
import jax.numpy as jnp
from jax import lax
from jax.experimental import pallas as pl
from jax.experimental.pallas import tpu as pltpu
from jax.experimental.pallas import tpu_sc as plsc

N = 50000
D = 256
G = 1024
GH = 2 * D

SEG_PER_TILE = G // 32
CHUNK = 80
NGRP = CHUNK // 16
NIDS = 50176
NSENT = (NIDS - N) // 16


def _bcast_lane(vec, lane):
    idx = jnp.full((16, 1), lane, jnp.int32)
    return lax.gather(
        vec, idx,
        dimension_numbers=lax.GatherDimensionNumbers(
            offset_dims=(), collapsed_slice_dims=(0,), start_index_map=(0,)),
        slice_sizes=(1,),
        mode=lax.GatherScatterMode.PROMISE_IN_BOUNDS)


def _sc_body(hv_hbm, ids_hbm, wg_hbm, bg_hbm, pw_hbm, pc_hbm,
             hvbuf, idsf, wgv, bgv, acc, cacc, sem):
    cid = lax.axis_index("c")
    sid = lax.axis_index("s")
    t = sid * 2 + cid
    iota16 = lax.iota(jnp.int32, 16)
    fzero = jnp.zeros((16,), jnp.float32)

    def zs(s, c0):
        for k in range(16):
            acc[s, pl.ds(k * 16, 16)] = fzero
        cacc[s, :] = fzero
        return c0

    lax.fori_loop(0, SEG_PER_TILE, zs, 0)

    pltpu.sync_copy(ids_hbm, idsf.at[pl.ds(0, N)])

    def sent(i, c0):
        idsf[pl.ds(N + i * 16, 16)] = jnp.full((16,), G, jnp.int32)
        return c0

    lax.fori_loop(0, NSENT, sent, 0)
    pltpu.sync_copy(wg_hbm, wgv)
    pltpu.sync_copy(bg_hbm, bgv)

    thr_lo = t * SEG_PER_TILE
    thr_hi = thr_lo + SEG_PER_TILE

    def cnt(i, carry):
        v = idsf[pl.ds(i * 16, 16)]
        cl, ch = carry
        cl = cl + jnp.where(v < thr_lo, 1.0, 0.0)
        ch = ch + jnp.where(v < thr_hi, 1.0, 0.0)
        return (cl, ch)

    clv, chv = lax.fori_loop(0, NIDS // 16, cnt, (fzero, fzero))
    lo = jnp.sum(clv).astype(jnp.int32)
    hi = jnp.sum(chv).astype(jnp.int32)
    seg0 = thr_lo

    wgs = [wgv[pl.ds(k * 16, 16)] for k in range(16)]
    bg0 = bgv[...]

    q0 = lo // CHUNK
    q1 = (hi + CHUNK - 1) // CHUNK

    @pl.when(q0 < q1)
    def _prime():
        pltpu.async_copy(hv_hbm.at[pl.ds(q0 * CHUNK, CHUNK)],
                         hvbuf.at[0], sem.at[0])

    def chunk(q, regs):
        off = q * CHUNK
        slot = (q - q0) % 2
        pltpu.make_async_copy(hv_hbm.at[pl.ds(0, CHUNK)],
                              hvbuf.at[slot], sem.at[slot]).wait()

        @pl.when(q + 1 < q1)
        def _next():
            pltpu.async_copy(hv_hbm.at[pl.ds((q + 1) * CHUNK, CHUNK)],
                             hvbuf.at[1 - slot], sem.at[1 - slot])

        def group(m, regs):
            base = off + m * 16
            vc = idsf[pl.ds(base, 16)]
            vn = idsf[pl.ds(base + 16, 16)]
            bcs = [_bcast_lane(vc, l) for l in range(16)]
            bcs.append(_bcast_lane(vn, 0))
            for l in range(16):
                jj = m * 16 + l
                rows = [hvbuf[slot, jj, pl.ds(k * 16, 16)]
                        for k in range(16)]
                prods = [rows[k] * wgs[k] for k in range(16)]
                for step in (8, 4, 2, 1):
                    prods = [prods[i2] + prods[i2 + step]
                             for i2 in range(step)]
                x = jnp.broadcast_to(jnp.sum(prods[0]), (16,)) + bg0
                gate = 1.0 / (1.0 + jnp.exp(-x))

                cur = bcs[l]
                changed = cur != bcs[l + 1]
                loc = cur - seg0
                mfl = changed & (loc >= 0) & (loc < SEG_PER_TILE)
                nregs = []
                for k in range(16):
                    r = regs[k] + rows[k] * gate
                    plsc.addupdate_scatter(
                        acc, [loc, iota16 + k * 16], r, mask=mfl)
                    nregs.append(jnp.where(changed, fzero, r))
                rg = regs[16] + gate
                plsc.addupdate_scatter(cacc, [loc, iota16], rg, mask=mfl)
                nregs.append(jnp.where(changed, fzero, rg))
                regs = tuple(nregs)
            return regs

        return lax.fori_loop(0, NGRP, group, regs)

    lax.fori_loop(q0, q1, chunk, (fzero,) * 17)

    pltpu.sync_copy(acc, pw_hbm.at[pl.ds(t * SEG_PER_TILE, SEG_PER_TILE)])
    pltpu.sync_copy(cacc, pc_hbm.at[pl.ds(t * SEG_PER_TILE, SEG_PER_TILE)])


def _final_body(pw_ref, pc_ref, wp_ref, bp_ref, out_ref):
    s = pw_ref[...]
    c = pc_ref[:, 0:1]
    out_ref[...] = lax.dot_general(
        s, wp_ref[...], (((1,), (1,)), ((), ())),
        preferred_element_type=jnp.float32) + c * bp_ref[...]


def kernel(hv, segment_ids, W_gate, b_gate, W_proj, b_proj):
    bp = b_proj.reshape(1, GH)
    ids = segment_ids.astype(jnp.int32)
    wg = W_gate.reshape(D)
    bg16 = jnp.broadcast_to(b_gate, (16,))

    mesh = plsc.VectorSubcoreMesh(core_axis_name="c", subcore_axis_name="s")
    pw, pc = pl.kernel(
        _sc_body,
        out_type=[
            jax.ShapeDtypeStruct((G, D), jnp.float32),
            jax.ShapeDtypeStruct((G, 16), jnp.float32),
        ],
        mesh=mesh,
        compiler_params=pltpu.CompilerParams(needs_layout_passes=False),
        scratch_types=[
            pltpu.VMEM((2, CHUNK, D), jnp.float32),
            pltpu.VMEM((NIDS,), jnp.int32),
            pltpu.VMEM((D,), jnp.float32),
            pltpu.VMEM((16,), jnp.float32),
            pltpu.VMEM((SEG_PER_TILE, D), jnp.float32),
            pltpu.VMEM((SEG_PER_TILE, 16), jnp.float32),
            pltpu.SemaphoreType.DMA((2,)),
        ],
    )(hv, ids, wg, bg16)

    out = pl.pallas_call(
        _final_body,
        grid=(1,),
        in_specs=[
            pl.BlockSpec((G, D), lambda i: (0, 0)),
            pl.BlockSpec((G, 16), lambda i: (0, 0)),
            pl.BlockSpec((GH, D), lambda i: (0, 0)),
            pl.BlockSpec((1, GH), lambda i: (0, 0)),
        ],
        out_specs=pl.BlockSpec((G, GH), lambda i: (0, 0)),
        out_shape=jax.ShapeDtypeStruct((G, GH), jnp.float32),
    )(pw, pc, W_proj, bp)
    return out
